# baseline (speedup 1.0000x reference)
import jax
import jax.numpy as jnp
from jax.experimental import pallas as pl
from jax.experimental.pallas import tpu as pltpu

_K = 16
_RB = 512
_RB2 = 1024


def _knn_conv_kernel(x_row_ref, x_full_ref, ws_ref, wn_ref, par_ref,
                     h_ref, psum_ref, dist_ref):
    rb = pl.program_id(1)
    n = x_full_ref.shape[1]
    x_row = x_row_ref[0]
    x_full = x_full_ref[0]

    xxr = jnp.sum(x_row * x_row, axis=1, keepdims=True)
    xxf = jnp.sum(x_full * x_full, axis=1, keepdims=True)
    xy = jax.lax.dot_general(-2.0 * x_row, x_full, (((1,), (1,)), ((), ())),
                             preferred_element_type=jnp.float32)
    dist = xy + (xxr + xxf.T)

    row_ids = rb * _RB + jax.lax.broadcasted_iota(jnp.int32, (_RB, n), 0)
    col_ids = jax.lax.broadcasted_iota(jnp.int32, (_RB, n), 1)
    dist = jnp.where(row_ids == col_ids, jnp.inf, dist)

    inf = jnp.float32(jnp.inf)
    dist_ref[...] = dist
    for _ in range(_K):
        cur = dist_ref[...]
        m = jnp.min(cur, axis=1, keepdims=True)
        dist_ref[...] = jnp.where(cur == m, inf, cur)
    msel = jnp.where(dist_ref[...] == inf, 1.0, 0.0).astype(jnp.float32)

    hn_full = jnp.dot(x_full, wn_ref[...], preferred_element_type=jnp.float32)
    agg = jnp.dot(msel, hn_full, preferred_element_type=jnp.float32)
    hs = jnp.dot(x_row, ws_ref[...], preferred_element_type=jnp.float32)
    hn_row = jnp.dot(x_row, wn_ref[...], preferred_element_type=jnp.float32)
    s2 = jnp.float32(1.0 / (16.0 + 1e-6))
    h = hs + (1.0 - s2) * hn_row + s2 * agg + par_ref[5:6, :]

    x2 = h * h
    num = par_ref[0:1, :] + par_ref[1:2, :] * h + par_ref[2:3, :] * x2
    den = 1.0 + jnp.abs(par_ref[3:4, :] * h + par_ref[4:5, :] * x2)
    h = num / (den + 1e-8)

    h_ref[0] = h
    psum_ref[0, 0:1, :] = jnp.sum(h, axis=0, keepdims=True)
    psum_ref[0, 1:2, :] = jnp.sum(h * h, axis=0, keepdims=True)


def _bn_kernel(h_ref, p_ref, par_ref, count_inv_ref, y_ref):
    sums = jnp.sum(p_ref[...], axis=0)
    cinv = count_inv_ref[0, 0]
    mean = sums[0:1, :] * cinv
    var = sums[1:2, :] * cinv - mean * mean
    inv = jax.lax.rsqrt(var + 1e-5)
    y_ref[...] = (h_ref[...] - mean) * (inv * par_ref[6:7, :]) + par_ref[7:8, :]


def kernel(x, weight_self, weight_neighbor, kan_a, kan_b, bias, bn_weight,
           bn_bias):
    b, n, d = x.shape
    nb = n // _RB

    packed = jnp.stack([kan_a[:, 0], kan_a[:, 1], kan_a[:, 2],
                        kan_b[:, 0], kan_b[:, 1], bias,
                        bn_weight, bn_bias], axis=0)

    h_kan, psums = pl.pallas_call(
        _knn_conv_kernel,
        grid=(b, nb),
        in_specs=[
            pl.BlockSpec((1, _RB, d), lambda bi, ri: (bi, ri, 0)),
            pl.BlockSpec((1, n, d), lambda bi, ri: (bi, 0, 0)),
            pl.BlockSpec((d, d), lambda bi, ri: (0, 0)),
            pl.BlockSpec((d, d), lambda bi, ri: (0, 0)),
            pl.BlockSpec((8, d), lambda bi, ri: (0, 0)),
        ],
        out_specs=[
            pl.BlockSpec((1, _RB, d), lambda bi, ri: (bi, ri, 0)),
            pl.BlockSpec((1, 8, d), lambda bi, ri: (bi * nb + ri, 0, 0)),
        ],
        out_shape=[
            jax.ShapeDtypeStruct((b, n, d), jnp.float32),
            jax.ShapeDtypeStruct((b * nb, 8, d), jnp.float32),
        ],
        compiler_params=pltpu.CompilerParams(
            dimension_semantics=("parallel", "parallel")),
        scratch_shapes=[pltpu.VMEM((_RB, n), jnp.float32)],
    )(x, x, weight_self, weight_neighbor, packed)

    count_inv = jnp.full((1, 1), 1.0 / (b * n), jnp.float32)
    h_flat = h_kan.reshape(b * n, d)
    y = pl.pallas_call(
        _bn_kernel,
        grid=(b * n // _RB2,),
        in_specs=[
            pl.BlockSpec((_RB2, d), lambda i: (i, 0)),
            pl.BlockSpec((b * nb, 8, d), lambda i: (0, 0, 0)),
            pl.BlockSpec((8, d), lambda i: (0, 0)),
            pl.BlockSpec((1, 1), lambda i: (0, 0), memory_space=pltpu.SMEM),
        ],
        out_specs=pl.BlockSpec((_RB2, d), lambda i: (i, 0)),
        out_shape=jax.ShapeDtypeStruct((b * n, d), jnp.float32),
    )(h_flat, psums, packed, count_inv)
    return y.reshape(b, n, d)


# speedup vs baseline: 1.0020x; 1.0020x over previous
import jax
import jax.numpy as jnp
from jax.experimental import pallas as pl
from jax.experimental.pallas import tpu as pltpu

_K = 16
_RB = 512
_RB2 = 1024


def _knn_conv_kernel(x_row_ref, x_full_ref, ws_ref, wn_ref, par_ref,
                     h_ref, psum_ref):
    rb = pl.program_id(1)
    n = x_full_ref.shape[1]
    x_row = x_row_ref[0]
    x_full = x_full_ref[0]

    xxr = jnp.sum(x_row * x_row, axis=1, keepdims=True)
    xxf = jnp.sum(x_full * x_full, axis=1, keepdims=True)
    xy = jax.lax.dot_general(-2.0 * x_row, x_full, (((1,), (1,)), ((), ())),
                             preferred_element_type=jnp.float32)
    dist = xy + (xxr + xxf.T)

    row_ids = rb * _RB + jax.lax.broadcasted_iota(jnp.int32, (_RB, n), 0)
    col_ids = jax.lax.broadcasted_iota(jnp.int32, (_RB, n), 1)
    dist = jnp.where(row_ids == col_ids, jnp.inf, dist)

    inf = jnp.float32(jnp.inf)
    m = jnp.min(dist, axis=1, keepdims=True)
    for _ in range(_K - 1):
        dist = jnp.where(dist == m, inf, dist)
        m = jnp.min(dist, axis=1, keepdims=True)
    msel = jnp.where((dist == inf) | (dist == m), 1.0, 0.0
                     ).astype(jnp.float32)

    hn_full = jnp.dot(x_full, wn_ref[...], preferred_element_type=jnp.float32)
    agg = jnp.dot(msel, hn_full, preferred_element_type=jnp.float32)
    hs = jnp.dot(x_row, ws_ref[...], preferred_element_type=jnp.float32)
    hn_row = jnp.dot(x_row, wn_ref[...], preferred_element_type=jnp.float32)
    s2 = jnp.float32(1.0 / (16.0 + 1e-6))
    h = hs + (1.0 - s2) * hn_row + s2 * agg + par_ref[5:6, :]

    x2 = h * h
    num = par_ref[0:1, :] + par_ref[1:2, :] * h + par_ref[2:3, :] * x2
    den = 1.0 + jnp.abs(par_ref[3:4, :] * h + par_ref[4:5, :] * x2)
    h = num / (den + 1e-8)

    h_ref[0] = h
    psum_ref[0, 0:1, :] = jnp.sum(h, axis=0, keepdims=True)
    psum_ref[0, 1:2, :] = jnp.sum(h * h, axis=0, keepdims=True)


def _bn_kernel(h_ref, p_ref, par_ref, count_inv_ref, y_ref):
    sums = jnp.sum(p_ref[...], axis=0)
    cinv = count_inv_ref[0, 0]
    mean = sums[0:1, :] * cinv
    var = sums[1:2, :] * cinv - mean * mean
    inv = jax.lax.rsqrt(var + 1e-5)
    y_ref[...] = (h_ref[...] - mean) * (inv * par_ref[6:7, :]) + par_ref[7:8, :]


def kernel(x, weight_self, weight_neighbor, kan_a, kan_b, bias, bn_weight,
           bn_bias):
    b, n, d = x.shape
    nb = n // _RB

    packed = jnp.stack([kan_a[:, 0], kan_a[:, 1], kan_a[:, 2],
                        kan_b[:, 0], kan_b[:, 1], bias,
                        bn_weight, bn_bias], axis=0)

    h_kan, psums = pl.pallas_call(
        _knn_conv_kernel,
        grid=(b, nb),
        in_specs=[
            pl.BlockSpec((1, _RB, d), lambda bi, ri: (bi, ri, 0)),
            pl.BlockSpec((1, n, d), lambda bi, ri: (bi, 0, 0)),
            pl.BlockSpec((d, d), lambda bi, ri: (0, 0)),
            pl.BlockSpec((d, d), lambda bi, ri: (0, 0)),
            pl.BlockSpec((8, d), lambda bi, ri: (0, 0)),
        ],
        out_specs=[
            pl.BlockSpec((1, _RB, d), lambda bi, ri: (bi, ri, 0)),
            pl.BlockSpec((1, 8, d), lambda bi, ri: (bi * nb + ri, 0, 0)),
        ],
        out_shape=[
            jax.ShapeDtypeStruct((b, n, d), jnp.float32),
            jax.ShapeDtypeStruct((b * nb, 8, d), jnp.float32),
        ],
        compiler_params=pltpu.CompilerParams(
            dimension_semantics=("parallel", "parallel")),
    )(x, x, weight_self, weight_neighbor, packed)

    count_inv = jnp.full((1, 1), 1.0 / (b * n), jnp.float32)
    h_flat = h_kan.reshape(b * n, d)
    y = pl.pallas_call(
        _bn_kernel,
        grid=(b * n // _RB2,),
        in_specs=[
            pl.BlockSpec((_RB2, d), lambda i: (i, 0)),
            pl.BlockSpec((b * nb, 8, d), lambda i: (0, 0, 0)),
            pl.BlockSpec((8, d), lambda i: (0, 0)),
            pl.BlockSpec((1, 1), lambda i: (0, 0), memory_space=pltpu.SMEM),
        ],
        out_specs=pl.BlockSpec((_RB2, d), lambda i: (i, 0)),
        out_shape=jax.ShapeDtypeStruct((b * n, d), jnp.float32),
    )(h_flat, psums, packed, count_inv)
    return y.reshape(b, n, d)


# speedup vs baseline: 1.0286x; 1.0265x over previous
import jax
import jax.numpy as jnp
from jax.experimental import pallas as pl
from jax.experimental.pallas import tpu as pltpu

_K = 16
_RB = 1024
_RB2 = 1024


def _knn_conv_kernel(x_row_ref, x_full_ref, ws_ref, wn_ref, par_ref,
                     h_ref, psum_ref):
    rb = pl.program_id(1)
    n = x_full_ref.shape[1]
    x_row = x_row_ref[0]
    x_full = x_full_ref[0]

    xxr = jnp.sum(x_row * x_row, axis=1, keepdims=True)
    xxf = jnp.sum(x_full * x_full, axis=1, keepdims=True)
    xy = jax.lax.dot_general(-2.0 * x_row, x_full, (((1,), (1,)), ((), ())),
                             preferred_element_type=jnp.float32)
    dist = xy + (xxr + xxf.T)

    row_ids = rb * _RB + jax.lax.broadcasted_iota(jnp.int32, (_RB, n), 0)
    col_ids = jax.lax.broadcasted_iota(jnp.int32, (_RB, n), 1)
    dist = jnp.where(row_ids == col_ids, jnp.inf, dist)

    inf = jnp.float32(jnp.inf)
    m = jnp.min(dist, axis=1, keepdims=True)
    for _ in range(_K - 1):
        dist = jnp.where(dist == m, inf, dist)
        m = jnp.min(dist, axis=1, keepdims=True)
    msel = jnp.where((dist == inf) | (dist == m), 1.0, 0.0
                     ).astype(jnp.float32)

    hn_full = jnp.dot(x_full, wn_ref[...], preferred_element_type=jnp.float32)
    agg = jnp.dot(msel, hn_full, preferred_element_type=jnp.float32)
    hs = jnp.dot(x_row, ws_ref[...], preferred_element_type=jnp.float32)
    hn_row = jnp.dot(x_row, wn_ref[...], preferred_element_type=jnp.float32)
    s2 = jnp.float32(1.0 / (16.0 + 1e-6))
    h = hs + (1.0 - s2) * hn_row + s2 * agg + par_ref[5:6, :]

    x2 = h * h
    num = par_ref[0:1, :] + par_ref[1:2, :] * h + par_ref[2:3, :] * x2
    den = 1.0 + jnp.abs(par_ref[3:4, :] * h + par_ref[4:5, :] * x2)
    h = num / (den + 1e-8)

    h_ref[0] = h
    psum_ref[0, 0:1, :] = jnp.sum(h, axis=0, keepdims=True)
    psum_ref[0, 1:2, :] = jnp.sum(h * h, axis=0, keepdims=True)


def _bn_kernel(h_ref, p_ref, par_ref, count_inv_ref, y_ref):
    sums = jnp.sum(p_ref[...], axis=0)
    cinv = count_inv_ref[0, 0]
    mean = sums[0:1, :] * cinv
    var = sums[1:2, :] * cinv - mean * mean
    inv = jax.lax.rsqrt(var + 1e-5)
    y_ref[...] = (h_ref[...] - mean) * (inv * par_ref[6:7, :]) + par_ref[7:8, :]


def kernel(x, weight_self, weight_neighbor, kan_a, kan_b, bias, bn_weight,
           bn_bias):
    b, n, d = x.shape
    nb = n // _RB

    packed = jnp.stack([kan_a[:, 0], kan_a[:, 1], kan_a[:, 2],
                        kan_b[:, 0], kan_b[:, 1], bias,
                        bn_weight, bn_bias], axis=0)

    h_kan, psums = pl.pallas_call(
        _knn_conv_kernel,
        grid=(b, nb),
        in_specs=[
            pl.BlockSpec((1, _RB, d), lambda bi, ri: (bi, ri, 0)),
            pl.BlockSpec((1, n, d), lambda bi, ri: (bi, 0, 0)),
            pl.BlockSpec((d, d), lambda bi, ri: (0, 0)),
            pl.BlockSpec((d, d), lambda bi, ri: (0, 0)),
            pl.BlockSpec((8, d), lambda bi, ri: (0, 0)),
        ],
        out_specs=[
            pl.BlockSpec((1, _RB, d), lambda bi, ri: (bi, ri, 0)),
            pl.BlockSpec((1, 8, d), lambda bi, ri: (bi * nb + ri, 0, 0)),
        ],
        out_shape=[
            jax.ShapeDtypeStruct((b, n, d), jnp.float32),
            jax.ShapeDtypeStruct((b * nb, 8, d), jnp.float32),
        ],
        compiler_params=pltpu.CompilerParams(
            dimension_semantics=("parallel", "parallel")),
    )(x, x, weight_self, weight_neighbor, packed)

    count_inv = jnp.full((1, 1), 1.0 / (b * n), jnp.float32)
    h_flat = h_kan.reshape(b * n, d)
    y = pl.pallas_call(
        _bn_kernel,
        grid=(b * n // _RB2,),
        in_specs=[
            pl.BlockSpec((_RB2, d), lambda i: (i, 0)),
            pl.BlockSpec((b * nb, 8, d), lambda i: (0, 0, 0)),
            pl.BlockSpec((8, d), lambda i: (0, 0)),
            pl.BlockSpec((1, 1), lambda i: (0, 0), memory_space=pltpu.SMEM),
        ],
        out_specs=pl.BlockSpec((_RB2, d), lambda i: (i, 0)),
        out_shape=jax.ShapeDtypeStruct((b * n, d), jnp.float32),
    )(h_flat, psums, packed, count_inv)
    return y.reshape(b, n, d)
